# baseline (speedup 1.0000x reference)
import jax
import jax.numpy as jnp
from jax.experimental import pallas as pl
from jax.experimental.pallas import tpu as pltpu


_BLK = 4096

_SELU_ALPHA = 1.6732632423543772848170429916717
_SELU_SCALE = 1.0507009873554804934193349852946


def _selu(x):
    return _SELU_SCALE * jnp.where(x > 0, x, _SELU_ALPHA * (jnp.exp(x) - 1.0))


def _split_hi_lo(a):
    hi = a.astype(jnp.bfloat16)
    lo = (a - hi.astype(jnp.float32)).astype(jnp.bfloat16)
    return hi, lo


def _dot3(a, b, dn):
    a_hi, a_lo = _split_hi_lo(a)
    b_hi, b_lo = _split_hi_lo(b)
    d = lambda x, y: jax.lax.dot_general(
        x, y, dn, preferred_element_type=jnp.float32)
    return d(a_hi, b_hi) + d(a_hi, b_lo) + d(a_lo, b_hi)


_DN_T = (((0,), (0,)), ((), ()))
_DN_N = (((1,), (0,)), ((), ()))


def _make_body(n_paths_s):
    def _mlp_body(np_ref, a_ref, b_ref, c_ref, d_ref, w1_ref, b1_ref, w2_ref,
                  b2_ref, w3_ref, b3_ref, out_ref):
        blk = a_ref.shape[0]
        zp = (np_ref[0, 0] - n_paths_s).astype(jnp.float32)
        x5 = jnp.stack(
            [a_ref[...], b_ref[...], c_ref[...], d_ref[...],
             jnp.full((blk,), zp, jnp.float32)], axis=0)
        w1e = jnp.concatenate(
            [w1_ref[0:4, :], jnp.sum(w1_ref[4:, :], axis=0, keepdims=True)],
            axis=0)
        h = _dot3(x5, w1e, _DN_T)
        h = _selu(h + b1_ref[...])
        h = _dot3(h, w2_ref[...], _DN_N) + b2_ref[...]
        h = _selu(h)
        out_ref[...] = _dot3(h, w3_ref[...], _DN_N) + b3_ref[...]
    return _mlp_body


def kernel(links, paths, sequences, n_links, n_paths, link_capacity,
           tx_policies, tx_weights, bandwith, tos, packets, AvgPkS,
           arrival_time, duration, deadline, volume, pW, pU, pb, lW, lU, lb,
           rW1, rb1, rW2, rb2, rW3, rb3, mW1, mb1, mW2, mb2, mW3, mb3):
    n_paths_s = bandwith.shape[0]
    d = mW1.shape[0]
    ru = mW2.shape[0]
    np_arr = jnp.reshape(jnp.asarray(n_paths, jnp.int32), (1, 1))

    grid = (pl.cdiv(n_paths_s, _BLK),)
    fixed = lambda i: (0, 0)
    vec = pl.BlockSpec((_BLK,), lambda i: (i,))
    return pl.pallas_call(
        _make_body(n_paths_s),
        grid=grid,
        in_specs=[
            pl.BlockSpec((1, 1), fixed),
            vec, vec, vec, vec,
            pl.BlockSpec((d, ru), fixed),
            pl.BlockSpec((ru,), lambda i: (0,)),
            pl.BlockSpec((ru, ru), fixed),
            pl.BlockSpec((ru,), lambda i: (0,)),
            pl.BlockSpec((ru, 1), fixed),
            pl.BlockSpec((1,), lambda i: (0,)),
        ],
        out_specs=pl.BlockSpec((_BLK, 1), lambda i: (i, 0)),
        out_shape=jax.ShapeDtypeStruct((n_paths_s, 1), jnp.float32),
        compiler_params=pltpu.CompilerParams(
            dimension_semantics=("parallel",)),
    )(np_arr, arrival_time, duration, deadline, volume,
      mW1, mb1, mW2, mb2, mW3, mb3)


# speedup vs baseline: 1.1082x; 1.1082x over previous
import jax
import jax.numpy as jnp
from jax.experimental import pallas as pl
from jax.experimental.pallas import tpu as pltpu


_BLK = 5120

_SELU_ALPHA = 1.6732632423543772848170429916717
_SELU_SCALE = 1.0507009873554804934193349852946
_SELU_SA = _SELU_SCALE * _SELU_ALPHA


def _selu(x):
    return jnp.where(x > 0, _SELU_SCALE * x, _SELU_SA * jnp.exp(x) - _SELU_SA)


def _split_hi_lo(a):
    hi = a.astype(jnp.bfloat16)
    lo = (a - hi.astype(jnp.float32)).astype(jnp.bfloat16)
    return hi, lo


def _dot3(a, b, dn):
    a_hi, a_lo = _split_hi_lo(a)
    b_hi, b_lo = _split_hi_lo(b)
    d = lambda x, y: jax.lax.dot_general(
        x, y, dn, preferred_element_type=jnp.float32)
    return d(a_hi, b_hi) + d(a_hi, b_lo) + d(a_lo, b_hi)


_DN_T = (((0,), (0,)), ((), ()))
_DN_N = (((1,), (0,)), ((), ()))


def _mlp_body(zp_ref, a_ref, b_ref, c_ref, d_ref, w1_ref, b1_ref, w2_ref,
              b2_ref, w3t_ref, b3_ref, out_ref):
    blk = a_ref.shape[0]
    x5 = jnp.stack(
        [a_ref[...], b_ref[...], c_ref[...], d_ref[...],
         jnp.full((blk,), zp_ref[0, 0], jnp.float32)], axis=0)
    w1e = jnp.concatenate(
        [w1_ref[0:4, :], jnp.sum(w1_ref[4:, :], axis=0, keepdims=True)], axis=0)
    h = _dot3(x5, w1e, _DN_T)
    h = _selu(h + b1_ref[...])
    h = _dot3(h, w2_ref[...], _DN_N) + b2_ref[...]
    h = _selu(h)
    out_ref[...] = jnp.sum(h * w3t_ref[...], axis=1, keepdims=True) + b3_ref[...]


def kernel(links, paths, sequences, n_links, n_paths, link_capacity,
           tx_policies, tx_weights, bandwith, tos, packets, AvgPkS,
           arrival_time, duration, deadline, volume, pW, pU, pb, lW, lU, lb,
           rW1, rb1, rW2, rb2, rW3, rb3, mW1, mb1, mW2, mb2, mW3, mb3):
    n_paths_s = bandwith.shape[0]
    d = mW1.shape[0]
    ru = mW2.shape[0]
    zero_p = jnp.asarray(n_paths - n_paths_s, jnp.float32)

    grid = (pl.cdiv(n_paths_s, _BLK),)
    fixed = lambda i: (0, 0)
    vec = pl.BlockSpec((_BLK,), lambda i: (i,))
    return pl.pallas_call(
        _mlp_body,
        grid=grid,
        in_specs=[
            pl.BlockSpec((1, 1), fixed),
            vec, vec, vec, vec,
            pl.BlockSpec((d, ru), fixed),
            pl.BlockSpec((1, ru), fixed),
            pl.BlockSpec((ru, ru), fixed),
            pl.BlockSpec((1, ru), fixed),
            pl.BlockSpec((1, ru), fixed),
            pl.BlockSpec((1, 1), fixed),
        ],
        out_specs=pl.BlockSpec((_BLK, 1), lambda i: (i, 0)),
        out_shape=jax.ShapeDtypeStruct((n_paths_s, 1), jnp.float32),
        compiler_params=pltpu.CompilerParams(
            dimension_semantics=("parallel",)),
    )(zero_p.reshape(1, 1), arrival_time, duration, deadline, volume,
      mW1, mb1.reshape(1, ru), mW2, mb2.reshape(1, ru),
      mW3.reshape(1, ru), mb3.reshape(1, 1))


# speedup vs baseline: 1.9164x; 1.7293x over previous
import jax
import jax.numpy as jnp
from jax.experimental import pallas as pl
from jax.experimental.pallas import tpu as pltpu


_BLK = 5120

_SELU_ALPHA = 1.6732632423543772848170429916717
_SELU_SCALE = 1.0507009873554804934193349852946
_SELU_SA = _SELU_SCALE * _SELU_ALPHA


def _selu(x):
    return jnp.where(x > 0, _SELU_SCALE * x, _SELU_SA * jnp.exp(x) - _SELU_SA)


def _split_hi_lo(a):
    hi = a.astype(jnp.bfloat16)
    lo = (a - hi.astype(jnp.float32)).astype(jnp.bfloat16)
    return hi, lo


def _dot3(a, b, dn):
    a_hi, a_lo = _split_hi_lo(a)
    b_hi, b_lo = _split_hi_lo(b)
    d = lambda x, y: jax.lax.dot_general(
        x, y, dn, preferred_element_type=jnp.float32)
    return d(a_hi, b_hi) + d(a_hi, b_lo) + d(a_lo, b_hi)


_DN_T = (((0,), (0,)), ((), ()))
_DN_N = (((1,), (0,)), ((), ()))


def _dot1(a, b, dn):
    return jax.lax.dot_general(
        a.astype(jnp.bfloat16), b.astype(jnp.bfloat16), dn,
        preferred_element_type=jnp.float32)


def _mlp_body(zp_ref, a_ref, b_ref, c_ref, d_ref, w1_ref, b1_ref, w2_ref,
              b2_ref, w3t_ref, b3_ref, out_ref):
    blk = a_ref.shape[0]
    x5 = jnp.stack(
        [a_ref[...], b_ref[...], c_ref[...], d_ref[...],
         jnp.full((blk,), zp_ref[0, 0], jnp.float32)], axis=0)
    w1e = jnp.concatenate(
        [w1_ref[0:4, :], jnp.sum(w1_ref[4:, :], axis=0, keepdims=True)], axis=0)
    h = _dot1(x5, w1e, _DN_T)
    h = _selu(h + b1_ref[...])
    h = _dot1(h, w2_ref[...], _DN_N) + b2_ref[...]
    h = _selu(h)
    hb = h.astype(jnp.bfloat16).astype(jnp.float32)
    wb = w3t_ref[...].astype(jnp.bfloat16).astype(jnp.float32)
    out_ref[...] = jnp.sum(hb * wb, axis=1, keepdims=True) + b3_ref[...]


def kernel(links, paths, sequences, n_links, n_paths, link_capacity,
           tx_policies, tx_weights, bandwith, tos, packets, AvgPkS,
           arrival_time, duration, deadline, volume, pW, pU, pb, lW, lU, lb,
           rW1, rb1, rW2, rb2, rW3, rb3, mW1, mb1, mW2, mb2, mW3, mb3):
    n_paths_s = bandwith.shape[0]
    d = mW1.shape[0]
    ru = mW2.shape[0]
    zero_p = jnp.asarray(n_paths - n_paths_s, jnp.float32)

    grid = (pl.cdiv(n_paths_s, _BLK),)
    fixed = lambda i: (0, 0)
    vec = pl.BlockSpec((_BLK,), lambda i: (i,))
    return pl.pallas_call(
        _mlp_body,
        grid=grid,
        in_specs=[
            pl.BlockSpec((1, 1), fixed),
            vec, vec, vec, vec,
            pl.BlockSpec((d, ru), fixed),
            pl.BlockSpec((1, ru), fixed),
            pl.BlockSpec((ru, ru), fixed),
            pl.BlockSpec((1, ru), fixed),
            pl.BlockSpec((1, ru), fixed),
            pl.BlockSpec((1, 1), fixed),
        ],
        out_specs=pl.BlockSpec((_BLK, 1), lambda i: (i, 0)),
        out_shape=jax.ShapeDtypeStruct((n_paths_s, 1), jnp.float32),
        compiler_params=pltpu.CompilerParams(
            dimension_semantics=("parallel",)),
    )(zero_p.reshape(1, 1), arrival_time, duration, deadline, volume,
      mW1, mb1.reshape(1, ru), mW2, mb2.reshape(1, ru),
      mW3.reshape(1, ru), mb3.reshape(1, 1))
